# baseline (speedup 1.0000x reference)
import jax
import jax.numpy as jnp
from jax.experimental import pallas as pl
from jax.experimental.pallas import tpu as pltpu

_G = 1000
_BAND = 256
_GPAD = ((_G - 1) // 8) * 8 + _BAND


def _fused(batch_ref, x_ref, w1_ref, b1_ref, w2_ref, b2_ref, out_ref,
           acc_ref, s_ref):
    i = pl.program_id(0)
    nb = pl.num_programs(0)

    @pl.when(i == 0)
    def _init():
        acc_ref[...] = jnp.zeros_like(acc_ref)
        s_ref[...] = jnp.zeros_like(s_ref)

    x = x_ref[...]
    bn = x.shape[0]
    w2 = w2_ref[...]
    h = jnp.tanh(
        jax.lax.dot_general(x, w1_ref[...], (((1,), (0,)), ((), ())),
                            preferred_element_type=jnp.float32)
        + b1_ref[...]
    )
    l = jax.lax.dot_general(h, w2, (((1,), (0,)), ((), ())),
                            preferred_element_type=jnp.float32) + b2_ref[0, 0]
    m = jnp.sum(jnp.abs(w2)) + jnp.abs(b2_ref[0, 0])
    e = jnp.exp(l - m)
    eb = e.astype(jnp.bfloat16)
    weighted = (x * e).astype(jnp.bfloat16)

    g = batch_ref[0, 0, :]
    g0 = (batch_ref[0, 0, 0] // 8) * 8
    k = (batch_ref[0, 0, bn - 1] - g0) // _BAND + 1

    def body(b, _):
        start = g0 + b * _BAND
        rows = start + jax.lax.broadcasted_iota(jnp.int32, (_BAND, bn), 0)
        ohb = (rows == g[None, :]).astype(jnp.bfloat16)
        contrib = jax.lax.dot_general(
            ohb, weighted, (((1,), (0,)), ((), ())),
            preferred_element_type=jnp.float32)
        acc_ref[pl.ds(start, _BAND), :] += contrib
        sc = jax.lax.dot_general(
            ohb, eb, (((1,), (0,)), ((), ())),
            preferred_element_type=jnp.float32)
        s_ref[pl.ds(start, _BAND), :] += sc
        return 0

    jax.lax.fori_loop(0, k, body, 0)

    @pl.when(i == nb - 1)
    def _fin():
        out_ref[...] = acc_ref[:_G, :] / (s_ref[:_G, :] + 1e-16)


def kernel(x, batch, W1, b1, W2, b2):
    n, d = x.shape
    bn = 20000
    nb = n // bn
    batch3 = batch.astype(jnp.int32).reshape(nb, 1, bn)
    b1r = b1.reshape(1, d)
    b2r = b2.reshape(1, 1)
    return pl.pallas_call(
        _fused,
        grid=(nb,),
        in_specs=[
            pl.BlockSpec((1, 1, bn), lambda i: (i, 0, 0)),
            pl.BlockSpec((bn, d), lambda i: (i, 0)),
            pl.BlockSpec((d, d), lambda i: (0, 0)),
            pl.BlockSpec((1, d), lambda i: (0, 0)),
            pl.BlockSpec((d, 1), lambda i: (0, 0)),
            pl.BlockSpec((1, 1), lambda i: (0, 0)),
        ],
        out_specs=pl.BlockSpec((_G, d), lambda i: (0, 0)),
        out_shape=jax.ShapeDtypeStruct((_G, d), jnp.float32),
        scratch_shapes=[
            pltpu.VMEM((_GPAD, d), jnp.float32),
            pltpu.VMEM((_GPAD, 1), jnp.float32),
        ],
    )(batch3, x, W1, b1r, W2, b2r)


# speedup vs baseline: 1.5058x; 1.5058x over previous
import jax
import jax.numpy as jnp
from jax.experimental import pallas as pl
from jax.experimental.pallas import tpu as pltpu

_G = 1000
_D = 128
_BAND = 128
_GPAD = ((_G - 1) // 8) * 8 + _BAND


def _fused(batch_ref, x_ref, w1_ref, b1_ref, w2_ref, b2_ref, out_ref,
           acc_ref):
    i = pl.program_id(0)
    nb = pl.num_programs(0)

    @pl.when(i == 0)
    def _init():
        acc_ref[...] = jnp.zeros_like(acc_ref)

    x = x_ref[...]
    bn = x.shape[0]
    w2 = w2_ref[...]
    h = jnp.tanh(
        jax.lax.dot_general(x, w1_ref[...], (((1,), (0,)), ((), ())),
                            preferred_element_type=jnp.float32)
        + b1_ref[...]
    )
    l = jax.lax.dot_general(h, w2, (((1,), (0,)), ((), ())),
                            preferred_element_type=jnp.float32) + b2_ref[0, 0]
    m = jnp.sum(jnp.abs(w2)) + jnp.abs(b2_ref[0, 0])
    e = jnp.exp(l - m)
    eb = e.astype(jnp.bfloat16)
    weighted = jnp.concatenate(
        [(x * e).astype(jnp.bfloat16), eb], axis=1)

    g = batch_ref[0, 0, :]
    g0 = (batch_ref[0, 0, 0] // 8) * 8
    k = (batch_ref[0, 0, bn - 1] - g0) // _BAND + 1

    def body(b, _):
        start = g0 + b * _BAND
        rows = start + jax.lax.broadcasted_iota(jnp.int32, (_BAND, bn), 0)
        ohb = (rows == g[None, :]).astype(jnp.bfloat16)
        contrib = jax.lax.dot_general(
            ohb, weighted, (((1,), (0,)), ((), ())),
            preferred_element_type=jnp.float32)
        acc_ref[pl.ds(start, _BAND), :] += contrib
        return 0

    jax.lax.fori_loop(0, k, body, 0)

    @pl.when(i == nb - 1)
    def _fin():
        out_ref[...] = (acc_ref[:_G, :_D]
                        / (acc_ref[:_G, _D:_D + 1] + 1e-16))


def kernel(x, batch, W1, b1, W2, b2):
    n, d = x.shape
    bn = 10000
    nb = n // bn
    batch3 = batch.astype(jnp.int32).reshape(nb, 1, bn)
    b1r = b1.reshape(1, d)
    b2r = b2.reshape(1, 1)
    return pl.pallas_call(
        _fused,
        grid=(nb,),
        in_specs=[
            pl.BlockSpec((1, 1, bn), lambda i: (i, 0, 0)),
            pl.BlockSpec((bn, d), lambda i: (i, 0)),
            pl.BlockSpec((d, d), lambda i: (0, 0)),
            pl.BlockSpec((1, d), lambda i: (0, 0)),
            pl.BlockSpec((d, 1), lambda i: (0, 0)),
            pl.BlockSpec((1, 1), lambda i: (0, 0)),
        ],
        out_specs=pl.BlockSpec((_G, d), lambda i: (0, 0)),
        out_shape=jax.ShapeDtypeStruct((_G, d), jnp.float32),
        scratch_shapes=[
            pltpu.VMEM((_GPAD, d + 1), jnp.float32),
        ],
    )(batch3, x, W1, b1r, W2, b2r)
